# Initial kernel scaffold; baseline (speedup 1.0000x reference)
import jax
import jax.numpy as jnp
from jax.experimental import pallas as pl


def kernel(node_features, edge_features, edge_index, alpha, W1, b1, W2, b2, U, bU, V, bV, A, bA, B, bB, C, bC):
    raise NotImplementedError("write your pallas kernel here")


# speedup vs baseline: 1.7345x; 1.7345x over previous
import functools

import jax
import jax.numpy as jnp
from jax import lax
from jax.experimental import pallas as pl
from jax.experimental.pallas import tpu as pltpu
from jax.experimental.pallas import tpu_sc as plsc

N = 10000
E = 320000
DF = 128
DE = 16
UN = 128
L = 3
HALF = 64
NSUB = 16
CH = 128
NCHUNK = E // CH
NSTRIPE = 10
ROWS_PER_SUB = N // NSTRIPE
PREC = lax.Precision.HIGHEST

BMN = 2000
BME = 2000


def _mm_bias_body(x_ref, w_ref, b_ref, o_ref):
    o_ref[...] = jnp.dot(x_ref[...], w_ref[...], precision=PREC) + b_ref[...]


def _h0(node_features, W1, b1):
    return pl.pallas_call(
        _mm_bias_body,
        grid=(N // BMN,),
        in_specs=[pl.BlockSpec((BMN, DF), lambda i: (i, 0)),
                  pl.BlockSpec((DF, UN), lambda i: (0, 0)),
                  pl.BlockSpec((1, UN), lambda i: (0, 0))],
        out_specs=pl.BlockSpec((BMN, UN), lambda i: (i, 0)),
        out_shape=jax.ShapeDtypeStruct((N, UN), jnp.float32),
    )(node_features, W1, b1.reshape(1, UN))


def _e0_body(ef_ref, w2_ref, b2_ref, c_ref, bc_ref, e_ref, ec_ref):
    e0 = jnp.dot(ef_ref[...], w2_ref[...], precision=PREC) + b2_ref[...]
    e_ref[...] = e0
    r = jnp.dot(e0, c_ref[...], precision=PREC) + bc_ref[...]
    ec_ref[0] = r[:, :HALF]
    ec_ref[1] = r[:, HALF:]


def _e0(edge_features, W2, b2, C0, bC0):
    return pl.pallas_call(
        _e0_body,
        grid=(E // BME,),
        in_specs=[pl.BlockSpec((BME, DE), lambda i: (i, 0)),
                  pl.BlockSpec((DE, UN), lambda i: (0, 0)),
                  pl.BlockSpec((1, UN), lambda i: (0, 0)),
                  pl.BlockSpec((UN, UN), lambda i: (0, 0)),
                  pl.BlockSpec((1, UN), lambda i: (0, 0))],
        out_specs=[pl.BlockSpec((BME, UN), lambda i: (i, 0)),
                   pl.BlockSpec((2, BME, HALF), lambda i: (0, i, 0))],
        out_shape=[jax.ShapeDtypeStruct((E, UN), jnp.float32),
                   jax.ShapeDtypeStruct((2, E, HALF), jnp.float32)],
    )(edge_features, W2, b2.reshape(1, UN), C0, bC0.reshape(1, UN))


def _nodemats_body(h_ref, w_ref, b_ref, o_ref):
    o_ref[0, 0] = jnp.dot(h_ref[...], w_ref[0, 0], precision=PREC) + b_ref[0, 0]


def _nodemats(h, W3, b3):
    return pl.pallas_call(
        _nodemats_body,
        grid=(3, 2),
        in_specs=[pl.BlockSpec((N, UN), lambda w, c: (0, 0)),
                  pl.BlockSpec((1, 1, UN, HALF), lambda w, c: (w, c, 0, 0)),
                  pl.BlockSpec((1, 1, 1, HALF), lambda w, c: (w, c, 0, 0))],
        out_specs=pl.BlockSpec((1, 1, N, HALF), lambda w, c: (w, c, 0, 0)),
        out_shape=jax.ShapeDtypeStruct((3, 2, N, HALF), jnp.float32),
    )(h, W3, b3)


def _edge_body_enew(e_ref, rh_ref, c_ref, bc_ref, ec_ref, enew_ref):
    en = e_ref[...] + jnp.concatenate([rh_ref[0], rh_ref[1]], axis=1)
    enew_ref[...] = en
    r = jnp.dot(en, c_ref[...], precision=PREC) + bc_ref[...]
    ec_ref[0] = r[:, :HALF]
    ec_ref[1] = r[:, HALF:]


def _edge_body(e_ref, rh_ref, c_ref, bc_ref, ec_ref):
    en = e_ref[...] + jnp.concatenate([rh_ref[0], rh_ref[1]], axis=1)
    r = jnp.dot(en, c_ref[...], precision=PREC) + bc_ref[...]
    ec_ref[0] = r[:, :HALF]
    ec_ref[1] = r[:, HALF:]


def _edge_tc(e, rhat, Cl, bCl, want_enew):
    in_specs = [pl.BlockSpec((BME, UN), lambda i: (i, 0)),
                pl.BlockSpec((2, BME, HALF), lambda i: (0, i, 0)),
                pl.BlockSpec((UN, UN), lambda i: (0, 0)),
                pl.BlockSpec((1, UN), lambda i: (0, 0))]
    ec_spec = pl.BlockSpec((2, BME, HALF), lambda i: (0, i, 0))
    ec_shape = jax.ShapeDtypeStruct((2, E, HALF), jnp.float32)
    if want_enew:
        return pl.pallas_call(
            _edge_body_enew,
            grid=(E // BME,),
            in_specs=in_specs,
            out_specs=[ec_spec, pl.BlockSpec((BME, UN), lambda i: (i, 0))],
            out_shape=[ec_shape, jax.ShapeDtypeStruct((E, UN), jnp.float32)],
        )(e, rhat, Cl, bCl.reshape(1, UN))
    return pl.pallas_call(
        _edge_body,
        grid=(E // BME,),
        in_specs=in_specs,
        out_specs=[ec_spec],
        out_shape=[ec_shape],
    )(e, rhat, Cl, bCl.reshape(1, UN))[0]


def _nodeupd_body(h_ref, u_ref, bu_ref, agg_ref, den_ref, al_ref, o_ref):
    hu = jnp.dot(h_ref[...], u_ref[...], precision=PREC) + bu_ref[...]
    a = jnp.concatenate([agg_ref[0], agg_ref[1]], axis=1)
    d = jnp.concatenate([den_ref[0], den_ref[1]], axis=1) + 1e-6
    upd = jnp.maximum(hu + a / d, 0.0)
    o_ref[...] = h_ref[...] + al_ref[...] * upd


def _nodeupd(h, Ul, bUl, aggc, denc, alpha):
    return pl.pallas_call(
        _nodeupd_body,
        grid=(N // BMN,),
        in_specs=[pl.BlockSpec((BMN, UN), lambda i: (i, 0)),
                  pl.BlockSpec((UN, UN), lambda i: (0, 0)),
                  pl.BlockSpec((1, UN), lambda i: (0, 0)),
                  pl.BlockSpec((2, BMN, HALF), lambda i: (0, i, 0)),
                  pl.BlockSpec((2, BMN, HALF), lambda i: (0, i, 0)),
                  pl.BlockSpec((BMN, 1), lambda i: (i, 0))],
        out_specs=pl.BlockSpec((BMN, UN), lambda i: (i, 0)),
        out_shape=jax.ShapeDtypeStruct((N, UN), jnp.float32),
    )(h, Ul, bUl.reshape(1, UN), aggc, denc, alpha)


_SC_MESH = plsc.VectorSubcoreMesh(core_axis_name="c", subcore_axis_name="s")


def _make_sc_edge(write_rhat):
    out_type = []
    if write_rhat:
        out_type.append(jax.ShapeDtypeStruct((2, E, HALF), jnp.float32))
    out_type += [jax.ShapeDtypeStruct((2, N, HALF), jnp.float32),
                 jax.ShapeDtypeStruct((2, N, HALF), jnp.float32)]
    scratch = [
        pltpu.VMEM((1, 128), jnp.int32),
        pltpu.VMEM((1, 128), jnp.int32),
        pltpu.VMEM((1, 128), jnp.int32),
        pltpu.VMEM((CH, HALF), jnp.float32),
        pltpu.VMEM((CH, HALF), jnp.float32),
        pltpu.VMEM((CH, HALF), jnp.float32),
        pltpu.VMEM((CH, HALF), jnp.float32),
        pltpu.VMEM_SHARED((N, HALF), jnp.float32),
        pltpu.VMEM_SHARED((N, HALF), jnp.float32),
        pltpu.SemaphoreType.DMA,
    ]

    @functools.partial(
        pl.kernel, mesh=_SC_MESH, out_type=out_type, scratch_types=scratch,
        compiler_params=pltpu.CompilerParams(use_tc_tiling_on_sc=False))
    def sc_edge(eC_hbm, src2_hbm, dst2_hbm, tabA, tabB, tabV, zrows_hbm, *rest):
        if write_rhat:
            rhat_hbm, agg_hbm, den_hbm = rest[0], rest[1], rest[2]
            rest = rest[3:]
        else:
            rhat_hbm = None
            agg_hbm, den_hbm = rest[0], rest[1]
            rest = rest[2:]
        sidx, didx, dadj, gA, gB, gV, eCb, agg_sh, den_sh, sem = rest

        c = lax.axis_index("c")
        s = lax.axis_index("s")
        cN = c * N

        stripe = pl.ds(s * ROWS_PER_SUB, ROWS_PER_SUB)

        @pl.when(s < NSTRIPE)
        def _():
            pltpu.sync_copy(zrows_hbm, agg_sh.at[stripe])
            pltpu.sync_copy(zrows_hbm, den_sh.at[stripe])
        plsc.subcore_barrier()

        nk = (NCHUNK + NSUB - 1) // NSUB

        def chunk(k, carry):
            t = s + NSUB * k

            @pl.when(t < NCHUNK)
            def _():
                pltpu.sync_copy(src2_hbm.at[pl.ds(t, 1)], sidx)
                pltpu.sync_copy(dst2_hbm.at[pl.ds(t, 1)], didx)

                def adj(i, cy):
                    sl = pl.ds(i * 16, 16)
                    sidx[0, sl] = sidx[0, sl] + cN
                    dadj[0, sl] = didx[0, sl] + cN
                    return cy
                lax.fori_loop(0, 8, adj, 0)

                cps = [
                    pltpu.async_copy(tabA.at[sidx.at[0]], gA, sem),
                    pltpu.async_copy(tabB.at[dadj.at[0]], gB, sem),
                    pltpu.async_copy(tabV.at[sidx.at[0]], gV, sem),
                ]
                pltpu.sync_copy(eC_hbm.at[c, pl.ds(t * CH, CH)], eCb)
                for cp in cps:
                    cp.wait()

                def comp(r, cy):
                    for j in range(4):
                        sl = pl.ds(j * 16, 16)
                        ehat = gA[r, sl] + gB[r, sl] + eCb[r, sl]
                        gA[r, sl] = jnp.maximum(ehat, 0.0)
                        eta = 1.0 / (1.0 + jnp.exp(-ehat))
                        gB[r, sl] = eta
                        gV[r, sl] = eta * gV[r, sl]
                    return cy
                lax.fori_loop(0, CH, comp, 0)

                if write_rhat:
                    pltpu.sync_copy(gA, rhat_hbm.at[c, pl.ds(t * CH, CH)])
                pltpu.sync_copy(gV, agg_sh.at[didx.at[0]], add=True)
                pltpu.sync_copy(gB, den_sh.at[didx.at[0]], add=True)
            return carry

        lax.fori_loop(0, nk, chunk, 0)
        plsc.subcore_barrier()

        @pl.when(s < NSTRIPE)
        def _():
            pltpu.sync_copy(agg_sh.at[stripe], agg_hbm.at[c, stripe])
            pltpu.sync_copy(den_sh.at[stripe], den_hbm.at[c, stripe])

    return sc_edge


_SC_EDGE_RH = _make_sc_edge(True)
_SC_EDGE_LAST = _make_sc_edge(False)


def kernel(node_features, edge_features, edge_index, alpha,
           W1, b1, W2, b2, U, bU, V, bV, A, bA, B, bB, C, bC):
    src2 = edge_index[0].reshape(E // 128, 128)
    dst2 = edge_index[1].reshape(E // 128, 128)
    zrows = jnp.zeros((ROWS_PER_SUB, HALF), jnp.float32)

    h = _h0(node_features, W1, b1)
    e, eCcat = _e0(edge_features, W2, b2, C[0], bC[0])

    for l in range(L):
        W3 = (jnp.stack([A[l], B[l], V[l]])
              .reshape(3, UN, 2, HALF).transpose(0, 2, 1, 3))
        b3 = jnp.stack([bA[l], bB[l], bV[l]]).reshape(3, 2, 1, HALF)
        tabs = _nodemats(h, W3, b3)
        tabA = tabs[0].reshape(2 * N, HALF)
        tabB = tabs[1].reshape(2 * N, HALF)
        tabV = tabs[2].reshape(2 * N, HALF)

        if l < L - 1:
            rhat, aggc, denc = _SC_EDGE_RH(
                eCcat, src2, dst2, tabA, tabB, tabV, zrows)
        else:
            aggc, denc = _SC_EDGE_LAST(
                eCcat, src2, dst2, tabA, tabB, tabV, zrows)

        h = _nodeupd(h, U[l], bU[l], aggc, denc, alpha)

        if l == 0:
            eCcat, e = _edge_tc(e, rhat, C[1], bC[1], want_enew=True)
        elif l == 1:
            eCcat = _edge_tc(e, rhat, C[2], bC[2], want_enew=False)

    return h
